# baseline (speedup 1.0000x reference)
import functools

import jax
import jax.numpy as jnp
from jax import lax
from jax.experimental import pallas as pl
from jax.experimental.pallas import tpu as pltpu
from jax.experimental.pallas import tpu_sc as plsc

EMB = 64
PAIR = 2 * EMB
NC, NS = 2, 16
NW = NC * NS
CHUNK = 128
HALF_ROWS = 256


def _sc_gather3(tw, cw, ng, temb2, cemb2):
    b = tw.shape[0]
    bpw = b // NW
    nstages = bpw // HALF_ROWS
    nchunks = bpw // CHUNK
    tw3 = tw.reshape(NW, nchunks, CHUNK)
    cw3 = cw.reshape(NW, nchunks, CHUNK)
    ng3 = ng.reshape(NW, nchunks, CHUNK)

    mesh = plsc.VectorSubcoreMesh(core_axis_name="c", subcore_axis_name="s")

    @functools.partial(
        pl.kernel,
        mesh=mesh,
        out_type=[
            jax.ShapeDtypeStruct((b, PAIR), jnp.float32),
            jax.ShapeDtypeStruct((b, PAIR), jnp.float32),
            jax.ShapeDtypeStruct((b, PAIR), jnp.float32),
        ],
        scratch_types=[
            pltpu.VMEM((nchunks, CHUNK), jnp.int32),
            pltpu.VMEM((nchunks, CHUNK), jnp.int32),
            pltpu.VMEM((nchunks, CHUNK), jnp.int32),
            pltpu.VMEM((HALF_ROWS, PAIR), jnp.float32),
            pltpu.VMEM((HALF_ROWS, PAIR), jnp.float32),
            pltpu.SemaphoreType.DMA,
            pltpu.SemaphoreType.DMA,
            pltpu.SemaphoreType.DMA,
            pltpu.SemaphoreType.DMA,
        ],
    )
    def k(tw_hbm, cw_hbm, ng_hbm, temb_hbm, cemb_hbm,
          t_out, c_out, n_out,
          ti, ci, ni, bufa, bufb, gsa, gsb, wsa, wsb):
        wid = lax.axis_index("s") * NC + lax.axis_index("c")
        base = wid * bpw
        pltpu.sync_copy(tw_hbm.at[wid], ti)
        pltpu.sync_copy(cw_hbm.at[wid], ci)
        pltpu.sync_copy(ng_hbm.at[wid], ni)

        bufs = (bufa, bufb)
        gsems = (gsa, gsb)
        wsems = (wsa, wsb)
        stages = []
        for idx_ref, src, dst in ((ti, temb_hbm, t_out),
                                  (ci, cemb_hbm, c_out),
                                  (ni, temb_hbm, n_out)):
            for s in range(nstages):
                stages.append((idx_ref, src, dst, s * HALF_ROWS))

        gathers = [None, None]
        writes = [None, None]

        def fire_gather(i):
            idx_ref, src, _, off = stages[i]
            slot = i % 2
            cs = []
            for j in range(HALF_ROWS // CHUNK):
                cs.append(pltpu.async_copy(
                    src.at[idx_ref.at[(off + j * CHUNK) // CHUNK]],
                    bufs[slot].at[pl.ds(j * CHUNK, CHUNK)],
                    gsems[slot]))
            gathers[slot] = cs

        def fire_write(i):
            _, _, dst, off = stages[i]
            slot = i % 2
            writes[slot] = pltpu.async_copy(
                bufs[slot], dst.at[pl.ds(base + off, HALF_ROWS)], wsems[slot])

        fire_gather(0)
        for i in range(1, len(stages)):
            slot = i % 2
            if writes[slot] is not None:
                writes[slot].wait()
            fire_gather(i)
            prev = (i - 1) % 2
            for c in gathers[prev]:
                c.wait()
            fire_write(i - 1)
        last = (len(stages) - 1) % 2
        for c in gathers[last]:
            c.wait()
        fire_write(len(stages) - 1)
        writes[0].wait()
        writes[1].wait()

    return k(tw3, cw3, ng3, temb2, cemb2)


def _tc_loss(t_rows, c_rows, n_rows, tp, cp, np_):
    b = t_rows.shape[0]
    blk = 2048
    grid = b // blk

    def body(t_ref, c_ref, n_ref, tp_ref, cp_ref, np_ref, o_ref, acc_ref):
        i = pl.program_id(0)

        @pl.when(i == 0)
        def _():
            acc_ref[0] = 0.0

        def half(ref, p_ref):
            x = ref[...]
            p = p_ref[...] == 1
            return jnp.where(p, x[:, EMB:], x[:, :EMB])
        t = half(t_ref, tp_ref)
        c = half(c_ref, cp_ref)
        n = half(n_ref, np_ref)
        pos = jnp.sum(t * c, axis=1)
        neg = jnp.sum(n * c, axis=1)
        def ls(x):
            return jnp.minimum(x, 0.0) - jnp.log1p(jnp.exp(-jnp.abs(x)))
        acc_ref[0] += jnp.sum(ls(pos)) + jnp.sum(ls(-neg))

        @pl.when(i == grid - 1)
        def _():
            o_ref[...] = jnp.full((1, 1), -acc_ref[0], jnp.float32)

    row_spec = pl.BlockSpec((blk, PAIR), lambda i: (i, 0))
    par_spec = pl.BlockSpec((blk, 1), lambda i: (i, 0))
    out = pl.pallas_call(
        body,
        grid=(grid,),
        in_specs=[row_spec, row_spec, row_spec, par_spec, par_spec, par_spec],
        out_specs=pl.BlockSpec((1, 1), lambda i: (0, 0)),
        out_shape=jax.ShapeDtypeStruct((1, 1), jnp.float32),
        scratch_shapes=[pltpu.SMEM((1,), jnp.float32)],
    )(t_rows, c_rows, n_rows, tp, cp, np_)
    return out[0, 0]


def kernel(target_word, context_word, negative_example, target_emb, context_emb):
    b = target_word.shape[0]
    v = target_emb.shape[0]
    temb2 = target_emb.reshape(v // 2, PAIR)
    cemb2 = context_emb.reshape(v // 2, PAIR)
    tw = (target_word >> 1).astype(jnp.int32)
    cw = (context_word >> 1).astype(jnp.int32)
    ng = (negative_example >> 1).astype(jnp.int32)
    tp = (target_word & 1).astype(jnp.int32).reshape(b, 1)
    cp = (context_word & 1).astype(jnp.int32).reshape(b, 1)
    np_ = (negative_example & 1).astype(jnp.int32).reshape(b, 1)
    t_rows, c_rows, n_rows = _sc_gather3(tw, cw, ng, temb2, cemb2)
    return _tc_loss(t_rows, c_rows, n_rows, tp, cp, np_)


# speedup vs baseline: 1.0001x; 1.0001x over previous
import functools

import jax
import jax.numpy as jnp
from jax import lax
from jax.experimental import pallas as pl
from jax.experimental.pallas import tpu as pltpu
from jax.experimental.pallas import tpu_sc as plsc

EMB = 64
PAIR = 2 * EMB
NC, NS = 2, 16
NW = NC * NS
CHUNK = 128
HALF_ROWS = 256


def _sc_gather3(tp, cp, np_, temb2, cemb2):
    b = tp.shape[0]
    bpw = b // NW
    nstages = bpw // HALF_ROWS
    nchunks = bpw // CHUNK
    tp3 = tp.reshape(NW, nchunks, CHUNK)
    cp3 = cp.reshape(NW, nchunks, CHUNK)
    np3 = np_.reshape(NW, nchunks, CHUNK)

    mesh = plsc.VectorSubcoreMesh(core_axis_name="c", subcore_axis_name="s")

    @functools.partial(
        pl.kernel,
        mesh=mesh,
        out_type=[
            jax.ShapeDtypeStruct((b, PAIR), jnp.float32),
            jax.ShapeDtypeStruct((b, PAIR), jnp.float32),
            jax.ShapeDtypeStruct((b, PAIR), jnp.float32),
        ],
        scratch_types=[
            pltpu.VMEM((nchunks, CHUNK), jnp.int32),
            pltpu.VMEM((nchunks, CHUNK), jnp.int32),
            pltpu.VMEM((nchunks, CHUNK), jnp.int32),
            pltpu.VMEM((HALF_ROWS, PAIR), jnp.float32),
            pltpu.VMEM((HALF_ROWS, PAIR), jnp.float32),
            pltpu.SemaphoreType.DMA,
            pltpu.SemaphoreType.DMA,
            pltpu.SemaphoreType.DMA,
            pltpu.SemaphoreType.DMA,
        ],
    )
    def k(tp_hbm, cp_hbm, np_hbm, temb_hbm, cemb_hbm,
          t_out, c_out, n_out,
          ti, ci, ni, bufa, bufb, gsa, gsb, wsa, wsb):
        wid = lax.axis_index("s") * NC + lax.axis_index("c")
        base = wid * bpw
        pltpu.sync_copy(tp_hbm.at[wid], ti)
        pltpu.sync_copy(cp_hbm.at[wid], ci)
        pltpu.sync_copy(np_hbm.at[wid], ni)

        bufs = (bufa, bufb)
        gsems = (gsa, gsb)
        wsems = (wsa, wsb)
        stages = []
        for idx_ref, src, dst in ((ti, temb_hbm, t_out),
                                  (ci, cemb_hbm, c_out),
                                  (ni, temb_hbm, n_out)):
            for s in range(nstages):
                stages.append((idx_ref, src, dst, s * HALF_ROWS))

        gathers = [None, None]
        writes = [None, None]

        def fire_gather(i):
            idx_ref, src, _, off = stages[i]
            slot = i % 2
            cs = []
            for j in range(HALF_ROWS // CHUNK):
                cs.append(pltpu.async_copy(
                    src.at[idx_ref.at[(off + j * CHUNK) // CHUNK]],
                    bufs[slot].at[pl.ds(j * CHUNK, CHUNK)],
                    gsems[slot]))
            gathers[slot] = cs

        def fire_write(i):
            _, _, dst, off = stages[i]
            slot = i % 2
            writes[slot] = pltpu.async_copy(
                bufs[slot], dst.at[pl.ds(base + off, HALF_ROWS)], wsems[slot])

        fire_gather(0)
        for i in range(1, len(stages)):
            slot = i % 2
            if writes[slot] is not None:
                writes[slot].wait()
            fire_gather(i)
            prev = (i - 1) % 2
            for c in gathers[prev]:
                c.wait()
            fire_write(i - 1)
        last = (len(stages) - 1) % 2
        for c in gathers[last]:
            c.wait()
        fire_write(len(stages) - 1)
        writes[0].wait()
        writes[1].wait()

    return k(tp3, cp3, np3, temb2, cemb2)


def _tc_loss(t_pair, c_pair, n_pair, t_par, c_par, n_par):
    b = t_pair.shape[0]
    blk = 2048
    grid = b // blk

    def body(t_ref, c_ref, n_ref, tp_ref, cp_ref, np_ref, o_ref, acc_ref):
        i = pl.program_id(0)

        @pl.when(i == 0)
        def _():
            acc_ref[0] = 0.0

        def half(pair_ref, par_ref):
            p = pair_ref[...]
            sel = par_ref[...] > 0.5
            return jnp.where(sel, p[:, EMB:], p[:, :EMB])

        t = half(t_ref, tp_ref)
        c = half(c_ref, cp_ref)
        n = half(n_ref, np_ref)
        pos = jnp.sum(t * c, axis=1)
        neg = jnp.sum(n * c, axis=1)

        def ls(x):
            return jnp.minimum(x, 0.0) - jnp.log1p(jnp.exp(-jnp.abs(x)))

        acc_ref[0] += jnp.sum(ls(pos)) + jnp.sum(ls(-neg))

        @pl.when(i == grid - 1)
        def _():
            o_ref[...] = jnp.full((1, 1), -acc_ref[0], jnp.float32)

    pair_spec = pl.BlockSpec((blk, PAIR), lambda i: (i, 0))
    par_spec = pl.BlockSpec((blk, 1), lambda i: (i, 0))
    out = pl.pallas_call(
        body,
        grid=(grid,),
        in_specs=[pair_spec, pair_spec, pair_spec, par_spec, par_spec, par_spec],
        out_specs=pl.BlockSpec((1, 1), lambda i: (0, 0)),
        out_shape=jax.ShapeDtypeStruct((1, 1), jnp.float32),
        scratch_shapes=[pltpu.SMEM((1,), jnp.float32)],
    )(t_pair, c_pair, n_pair, t_par, c_par, n_par)
    return out[0, 0]


def kernel(target_word, context_word, negative_example, target_emb, context_emb):
    tw = target_word.astype(jnp.int32)
    cw = context_word.astype(jnp.int32)
    ng = negative_example.astype(jnp.int32)
    temb2 = target_emb.reshape(-1, PAIR)
    cemb2 = context_emb.reshape(-1, PAIR)
    t_pair, c_pair, n_pair = _sc_gather3(tw >> 1, cw >> 1, ng >> 1,
                                         temb2, cemb2)
    t_par = (tw & 1).astype(jnp.float32)[:, None]
    c_par = (cw & 1).astype(jnp.float32)[:, None]
    n_par = (ng & 1).astype(jnp.float32)[:, None]
    return _tc_loss(t_pair, c_pair, n_pair, t_par, c_par, n_par)


# speedup vs baseline: 1.0011x; 1.0010x over previous
import functools

import jax
import jax.numpy as jnp
from jax import lax
from jax.experimental import pallas as pl
from jax.experimental.pallas import tpu as pltpu
from jax.experimental.pallas import tpu_sc as plsc

EMB = 64
PAIR = 2 * EMB
NC, NS = 2, 16
NW = NC * NS
CHUNK = 128
HALF_ROWS = 256


def _sc_gather3(tw, cw, ng, temb2, cemb2):
    b = tw.shape[0]
    bpw = b // NW
    nstages = bpw // HALF_ROWS
    nchunks = bpw // CHUNK
    tw3 = tw.reshape(NW, nchunks, CHUNK)
    cw3 = cw.reshape(NW, nchunks, CHUNK)
    ng3 = ng.reshape(NW, nchunks, CHUNK)

    mesh = plsc.VectorSubcoreMesh(core_axis_name="c", subcore_axis_name="s")

    @functools.partial(
        pl.kernel,
        mesh=mesh,
        out_type=[
            jax.ShapeDtypeStruct((b, PAIR), jnp.float32),
            jax.ShapeDtypeStruct((b, PAIR), jnp.float32),
            jax.ShapeDtypeStruct((b, PAIR), jnp.float32),
        ],
        scratch_types=[
            pltpu.VMEM((nchunks, CHUNK), jnp.int32),
            pltpu.VMEM((nchunks, CHUNK), jnp.int32),
            pltpu.VMEM((nchunks, CHUNK), jnp.int32),
            pltpu.VMEM((HALF_ROWS, PAIR), jnp.float32),
            pltpu.VMEM((HALF_ROWS, PAIR), jnp.float32),
            pltpu.SemaphoreType.DMA,
            pltpu.SemaphoreType.DMA,
            pltpu.SemaphoreType.DMA,
            pltpu.SemaphoreType.DMA,
        ],
    )
    def k(tw_hbm, cw_hbm, ng_hbm, temb_hbm, cemb_hbm,
          t_out, c_out, n_out,
          ti, ci, ni, bufa, bufb, gsa, gsb, wsa, wsb):
        wid = lax.axis_index("s") * NC + lax.axis_index("c")
        base = wid * bpw
        pltpu.sync_copy(tw_hbm.at[wid], ti)
        pltpu.sync_copy(cw_hbm.at[wid], ci)
        pltpu.sync_copy(ng_hbm.at[wid], ni)

        bufs = (bufa, bufb)
        gsems = (gsa, gsb)
        wsems = (wsa, wsb)
        stages = []
        for idx_ref, src, dst in ((ti, temb_hbm, t_out),
                                  (ci, cemb_hbm, c_out),
                                  (ni, temb_hbm, n_out)):
            for s in range(nstages):
                stages.append((idx_ref, src, dst, s * HALF_ROWS))

        gathers = [None, None]
        writes = [None, None]

        def fire_gather(i):
            idx_ref, src, _, off = stages[i]
            slot = i % 2
            cs = []
            for j in range(HALF_ROWS // CHUNK):
                cs.append(pltpu.async_copy(
                    src.at[idx_ref.at[(off + j * CHUNK) // CHUNK]],
                    bufs[slot].at[pl.ds(j * CHUNK, CHUNK)],
                    gsems[slot]))
            gathers[slot] = cs

        def fire_write(i):
            _, _, dst, off = stages[i]
            slot = i % 2
            writes[slot] = pltpu.async_copy(
                bufs[slot], dst.at[pl.ds(base + off, HALF_ROWS)], wsems[slot])

        fire_gather(0)
        for i in range(1, len(stages)):
            slot = i % 2
            if writes[slot] is not None:
                writes[slot].wait()
            fire_gather(i)
            prev = (i - 1) % 2
            for c in gathers[prev]:
                c.wait()
            fire_write(i - 1)
        last = (len(stages) - 1) % 2
        for c in gathers[last]:
            c.wait()
        fire_write(len(stages) - 1)
        writes[0].wait()
        writes[1].wait()

    return k(tw3, cw3, ng3, temb2, cemb2)


def _tc_loss(t_pairs, c_pairs, n_pairs, tp, cp, np_):
    b = t_pairs.shape[0]
    blk = 2048
    grid = b // blk

    def body(t_ref, c_ref, n_ref, tp_ref, cp_ref, np_ref, o_ref, acc_ref):
        i = pl.program_id(0)

        @pl.when(i == 0)
        def _():
            acc_ref[0] = 0.0

        def pick(pair_ref, par_ref):
            v = pair_ref[...]
            odd = par_ref[...] != 0
            return jnp.where(odd, v[:, EMB:], v[:, :EMB])

        t = pick(t_ref, tp_ref)
        c = pick(c_ref, cp_ref)
        n = pick(n_ref, np_ref)
        pos = jnp.sum(t * c, axis=1)
        neg = jnp.sum(n * c, axis=1)

        def ls(x):
            return jnp.minimum(x, 0.0) - jnp.log1p(jnp.exp(-jnp.abs(x)))

        acc_ref[0] += jnp.sum(ls(pos)) + jnp.sum(ls(-neg))

        @pl.when(i == grid - 1)
        def _():
            o_ref[...] = jnp.full((1, 1), -acc_ref[0], jnp.float32)

    row_spec = pl.BlockSpec((blk, PAIR), lambda i: (i, 0))
    par_spec = pl.BlockSpec((blk, 1), lambda i: (i, 0))
    out = pl.pallas_call(
        body,
        grid=(grid,),
        in_specs=[row_spec, row_spec, row_spec, par_spec, par_spec, par_spec],
        out_specs=pl.BlockSpec((1, 1), lambda i: (0, 0)),
        out_shape=jax.ShapeDtypeStruct((1, 1), jnp.float32),
        scratch_shapes=[pltpu.SMEM((1,), jnp.float32)],
    )(t_pairs, c_pairs, n_pairs, tp, cp, np_)
    return out[0, 0]


def kernel(target_word, context_word, negative_example, target_emb, context_emb):
    tw = target_word.astype(jnp.int32)
    cw = context_word.astype(jnp.int32)
    ng = negative_example.astype(jnp.int32)
    nvocab = target_emb.shape[0]
    temb2 = target_emb.reshape(nvocab // 2, PAIR)
    cemb2 = context_emb.reshape(nvocab // 2, PAIR)
    t_pairs, c_pairs, n_pairs = _sc_gather3(tw >> 1, cw >> 1, ng >> 1,
                                            temb2, cemb2)
    tp = (tw & 1).reshape(-1, 1)
    cp = (cw & 1).reshape(-1, 1)
    np_ = (ng & 1).reshape(-1, 1)
    return _tc_loss(t_pairs, c_pairs, n_pairs, tp, cp, np_)
